# Initial kernel scaffold; baseline (speedup 1.0000x reference)
import jax
import jax.numpy as jnp
from jax.experimental import pallas as pl


def kernel(attn_bias, spatial_pos, spatial_table, virtual_table):
    raise NotImplementedError("write your pallas kernel here")


# speedup vs baseline: 1.8696x; 1.8696x over previous
import jax
import jax.numpy as jnp
from jax import lax
from jax.experimental import pallas as pl
from jax.experimental.pallas import tpu as pltpu
from jax.experimental.pallas import tpu_sc as plsc

H = 32
S = 513
P = S * S
C = 8208
NFULL = P // C
TAIL = P - NFULL * C
PPAD = (NFULL + 1) * C
TW = 520
HPT = 8


def _sc_body(ab_hbm, idx_hbm, tbl_hbm, out_hbm, tbl_v, idx_v, ab_v, out_v):
    cid = lax.axis_index("c")
    sid = lax.axis_index("s")
    wid = sid * 2 + cid
    b = wid // 4
    hbase = (wid % 4) * HPT

    pltpu.sync_copy(tbl_hbm, tbl_v)

    def do_chunk(blk, nvec, nout):
        off = blk * C
        pltpu.sync_copy(idx_hbm.at[b, pl.ds(off, C)], idx_v)
        pltpu.sync_copy(ab_hbm.at[b, pl.ds(off, C)], ab_v)
        for h8 in range(HPT):
            hh = hbase + h8
            hoff = jnp.full((16,), hh * TW, jnp.int32)

            def vec_body(iv, _):
                p0 = iv * 16
                idxv = idx_v[pl.ds(p0, 16)]
                abv = ab_v[pl.ds(p0, 16)]
                val = plsc.load_gather(tbl_v, [hoff + idxv])
                out_v[pl.ds(p0, 16)] = abv + abv + val
                return 0

            lax.fori_loop(0, nvec, vec_body, 0)
            pltpu.sync_copy(out_v.at[pl.ds(0, nout)],
                            out_hbm.at[b, hh, pl.ds(off, nout)])

    def blk_body(blk, _):
        do_chunk(blk, C // 16, C)
        return 0

    lax.fori_loop(0, NFULL, blk_body, 0)
    do_chunk(NFULL, (TAIL + 15) // 16, TAIL)


def kernel(attn_bias, spatial_pos, spatial_table, virtual_table):
    B = attn_bias.shape[0]
    sp = spatial_pos.astype(jnp.int32)
    pidx = jnp.pad(sp, ((0, 0), (1, 0), (1, 0)), constant_values=512)
    pidx = jnp.pad(pidx.reshape(B, P), ((0, 0), (0, PPAD - P)))
    ab = jnp.pad(attn_bias.reshape(B, P), ((0, 0), (0, PPAD - P)))
    tbl = jnp.concatenate([spatial_table, virtual_table], axis=0)
    tblT = jnp.pad(tbl.T, ((0, 0), (0, TW - S))).reshape(H * TW)

    mesh = plsc.VectorSubcoreMesh(core_axis_name="c", subcore_axis_name="s")
    out = pl.kernel(
        _sc_body,
        mesh=mesh,
        compiler_params=pltpu.CompilerParams(use_tc_tiling_on_sc=False,
                                             needs_layout_passes=False),
        out_type=jax.ShapeDtypeStruct((B, H, P), jnp.float32),
        scratch_types=[
            pltpu.VMEM((H * TW,), jnp.float32),
            pltpu.VMEM((C,), jnp.int32),
            pltpu.VMEM((C,), jnp.float32),
            pltpu.VMEM((C,), jnp.float32),
        ],
    )(ab, pidx, tblT)
    return out.reshape(B, H, S, S)


# speedup vs baseline: 2.1755x; 1.1636x over previous
import jax
import jax.numpy as jnp
from jax import lax
from jax.experimental import pallas as pl
from jax.experimental.pallas import tpu as pltpu
from jax.experimental.pallas import tpu_sc as plsc

H = 32
S = 513
P = S * S
C = 4096
NFULL = P // C
TAIL = P - NFULL * C
NCHUNK = NFULL + 1
PPAD = (NCHUNK + 2) * C
TW = 520
HPT = 8


def _sc_body(ab_hbm, idx_hbm, tbl_hbm, out_hbm, *rest):
    tbl_v = rest[0]
    idx_v = rest[1:3]
    ab_v = rest[3:5]
    out_v = (rest[5:13], rest[13:21])
    in_sem = rest[21:23]
    out_sem = rest[23:25]
    cid = lax.axis_index("c")
    sid = lax.axis_index("s")
    wid = sid * 2 + cid
    b = wid // 4
    hbase = (wid % 4) * HPT
    hoffs = [(hbase + h8) * TW for h8 in range(HPT)]

    pltpu.sync_copy(tbl_hbm, tbl_v)

    def fire_in(blk, s):
        off = blk * C
        pltpu.async_copy(idx_hbm.at[b, pl.ds(off, C)], idx_v[s], in_sem[s])
        pltpu.async_copy(ab_hbm.at[b, pl.ds(off, C)], ab_v[s], in_sem[s])

    def wait_in(blk, s):
        off = blk * C
        pltpu.make_async_copy(idx_hbm.at[b, pl.ds(off, C)], idx_v[s],
                              in_sem[s]).wait()
        pltpu.make_async_copy(ab_hbm.at[b, pl.ds(off, C)], ab_v[s],
                              in_sem[s]).wait()

    def fire_out(blk, s, n):
        off = blk * C
        for h8 in range(HPT):
            pltpu.async_copy(out_v[s][h8].at[pl.ds(0, n)],
                             out_hbm.at[b, hbase + h8, pl.ds(off, n)],
                             out_sem[s])

    def wait_out(blk, s, n):
        off = blk * C
        for h8 in range(HPT):
            pltpu.make_async_copy(out_v[s][h8].at[pl.ds(0, n)],
                                  out_hbm.at[b, hbase + h8, pl.ds(off, n)],
                                  out_sem[s]).wait()

    def compute(s, nvec):
        @plsc.parallel_loop(0, nvec * 16, step=16, unroll=4)
        def vb(p0):
            idxv = idx_v[s][pl.ds(p0, 16)]
            abv = ab_v[s][pl.ds(p0, 16)]
            base = abv + abv
            for h8 in range(HPT):
                val = plsc.load_gather(tbl_v, [idxv + hoffs[h8]])
                out_v[s][h8][pl.ds(p0, 16)] = base + val

    fire_in(0, 0)
    fire_in(1, 1)
    wait_in(0, 0)
    compute(0, C // 16)
    fire_out(0, 0, C)
    fire_in(2, 0)
    wait_in(1, 1)
    compute(1, C // 16)
    fire_out(1, 1, C)
    fire_in(3, 1)

    def k_body(k, _):
        for s in range(2):
            blk = 2 * k + s
            wait_in(blk, s)
            wait_out(blk - 2, s, C)
            compute(s, C // 16)
            fire_out(blk, s, C)
            fire_in(blk + 2, s)
        return 0

    lax.fori_loop(1, NFULL // 2, k_body, 0)

    wait_in(NFULL, 0)
    wait_out(NFULL - 2, 0, C)
    compute(0, (TAIL + 15) // 16)
    fire_out(NFULL, 0, TAIL)
    wait_out(NFULL - 1, 1, C)
    wait_out(NFULL, 0, TAIL)


def kernel(attn_bias, spatial_pos, spatial_table, virtual_table):
    B = attn_bias.shape[0]
    sp = spatial_pos.astype(jnp.int32)
    pidx = jnp.pad(sp, ((0, 0), (1, 0), (1, 0)), constant_values=512)
    pidx = jnp.pad(pidx.reshape(B, P), ((0, 0), (0, PPAD - P)))
    ab = jnp.pad(attn_bias.reshape(B, P), ((0, 0), (0, PPAD - P)))
    tbl = jnp.concatenate([spatial_table, virtual_table], axis=0)
    tblT = jnp.pad(tbl.T, ((0, 0), (0, TW - S))).reshape(H * TW)

    mesh = plsc.VectorSubcoreMesh(core_axis_name="c", subcore_axis_name="s")
    out = pl.kernel(
        _sc_body,
        mesh=mesh,
        compiler_params=pltpu.CompilerParams(use_tc_tiling_on_sc=False,
                                             needs_layout_passes=False),
        out_type=jax.ShapeDtypeStruct((B, H, P), jnp.float32),
        scratch_types=[
            pltpu.VMEM((H * TW,), jnp.float32),
            pltpu.VMEM((C,), jnp.int32),
            pltpu.VMEM((C,), jnp.int32),
            pltpu.VMEM((C,), jnp.float32),
            pltpu.VMEM((C,), jnp.float32),
        ] + [pltpu.VMEM((C,), jnp.float32)] * (2 * HPT) + [
            pltpu.SemaphoreType.DMA,
            pltpu.SemaphoreType.DMA,
            pltpu.SemaphoreType.DMA,
            pltpu.SemaphoreType.DMA,
        ],
    )(ab, pidx, tblT)
    return out.reshape(B, H, S, S)
